# baseline (speedup 1.0000x reference)
import functools

import jax
import jax.numpy as jnp
from jax import lax
from jax.experimental import pallas as pl
from jax.experimental.pallas import tpu as pltpu
from jax.experimental.pallas import tpu_sc as plsc

NUM_NODES = 100000
D = 16
NC = 2
NS = 16
L = 16
KB = 50
K = KB * 128
NBUF = 2


def _add_body(p_ref, o_ref):
    o_ref[...] = p_ref[0] + p_ref[1]


@functools.lru_cache(maxsize=None)
def _sc_scatter(E):
    nblk = E // 128
    blk_sc = nblk // NC
    iters = blk_sc // KB
    steps = iters // NBUF
    e_sc = E // NC
    groups = K // L

    @functools.partial(
        pl.kernel,
        out_type=jax.ShapeDtypeStruct((NC, NS, NUM_NODES), jnp.float32),
        mesh=plsc.VectorSubcoreMesh(core_axis_name="c", subcore_axis_name="s"),
        compiler_params=pltpu.CompilerParams(
            use_tc_tiling_on_sc=False, needs_layout_passes=False),
        scratch_types=[
            pltpu.VMEM((NUM_NODES,), jnp.float32),
            pltpu.VMEM((NBUF, K), jnp.int32),
            pltpu.VMEM((NBUF, KB, 128), jnp.float32),
            pltpu.SemaphoreType.DMA((NBUF,)),
        ],
    )
    def k(idx1_hbm, attr1_hbm, idx2_hbm, attr2_hbm, out_hbm,
          acc, idx_v, vals_v, sems):
        cid = lax.axis_index("c")
        sid = lax.axis_index("s")
        fi = sid // 8
        f8 = sid % 8

        zeros16 = jnp.zeros((L,), jnp.float32)

        def zbody(t, carry):
            acc[pl.ds(t * L, L)] = zeros16
            return carry
        lax.fori_loop(0, NUM_NODES // L, zbody, 0)

        def issue(idx_hbm, attr_hbm, it, b):
            pltpu.async_copy(
                idx_hbm.at[pl.ds(cid * e_sc + it * K, K)],
                idx_v.at[b], sems.at[b])
            pltpu.async_copy(
                attr_hbm.at[fi, pl.ds(cid * blk_sc + it * KB, KB), f8],
                vals_v.at[b], sems.at[b])

        def drain(idx_hbm, attr_hbm, b):
            pltpu.make_async_copy(
                idx_hbm.at[pl.ds(0, K)], idx_v.at[b], sems.at[b]).wait()
            pltpu.make_async_copy(
                attr_hbm.at[0, pl.ds(0, KB), 0], vals_v.at[b],
                sems.at[b]).wait()

        def run(idx_hbm, attr_hbm):
            for b in range(NBUF):
                issue(idx_hbm, attr_hbm, b, b)

            def body(step, carry):
                for b in range(NBUF):
                    drain(idx_hbm, attr_hbm, b)

                    @plsc.parallel_loop(0, groups, 1, unroll=8)
                    def _(g):
                        gi = idx_v[b, pl.ds(g * L, L)]
                        gv = vals_v[b, g >> 3, pl.ds((g & 7) * L, L)]
                        plsc.addupdate_scatter(acc, [gi], gv)

                    @pl.when(step < steps - 1)
                    def _():
                        issue(idx_hbm, attr_hbm, (step + 1) * NBUF + b, b)
                return carry
            lax.fori_loop(0, steps, body, 0)

        run(idx1_hbm, attr1_hbm)
        run(idx2_hbm, attr2_hbm)

        pltpu.sync_copy(acc, out_hbm.at[cid, sid])

    return k


@functools.lru_cache(maxsize=None)
def _tc_add():
    blk = 6400
    return pl.pallas_call(
        _add_body,
        grid=(pl.cdiv(NUM_NODES, blk),),
        in_specs=[pl.BlockSpec((2, D, blk), lambda i: (0, 0, i))],
        out_specs=pl.BlockSpec((D, blk), lambda i: (0, i)),
        out_shape=jax.ShapeDtypeStruct((D, NUM_NODES), jnp.float32),
    )


def kernel(edge_attr, edge_attr2, edge_index, edge_index2, num_nodes):
    E = edge_attr.shape[0]
    nblk = E // 128

    def as_storage_view(attr):
        return attr.T.reshape(2, 8, nblk, 128).transpose(0, 2, 1, 3)

    idx1 = edge_index[0].astype(jnp.int32)
    idx2 = edge_index2[0].astype(jnp.int32)
    attr1 = as_storage_view(edge_attr)
    attr2 = as_storage_view(edge_attr2)

    partial = _sc_scatter(E)(idx1, attr1, idx2, attr2)
    out_t = _tc_add()(partial)
    return out_t.T


# speedup vs baseline: 1.0703x; 1.0703x over previous
import functools

import jax
import jax.numpy as jnp
from jax import lax
from jax.experimental import pallas as pl
from jax.experimental.pallas import tpu as pltpu
from jax.experimental.pallas import tpu_sc as plsc

NUM_NODES = 100000
D = 16
NC = 2
NS = 16
L = 16
KB = 25
K = KB * 128
NBUF = 4


def _add_body(p_ref, o_ref):
    o_ref[...] = p_ref[0] + p_ref[1]


@functools.lru_cache(maxsize=None)
def _sc_scatter(E):
    nblk = E // 128
    blk_sc = nblk // NC
    iters = blk_sc // KB
    steps = iters // NBUF
    e_sc = E // NC
    groups = K // L

    @functools.partial(
        pl.kernel,
        out_type=jax.ShapeDtypeStruct((NC, NS, NUM_NODES), jnp.float32),
        mesh=plsc.VectorSubcoreMesh(core_axis_name="c", subcore_axis_name="s"),
        compiler_params=pltpu.CompilerParams(
            use_tc_tiling_on_sc=False, needs_layout_passes=False),
        scratch_types=[
            pltpu.VMEM((NUM_NODES,), jnp.float32),
            pltpu.VMEM((NBUF, K), jnp.int32),
            pltpu.VMEM((NBUF, KB, 128), jnp.float32),
            pltpu.SemaphoreType.DMA((NBUF,)),
        ],
    )
    def k(idx1_hbm, attr1_hbm, idx2_hbm, attr2_hbm, out_hbm,
          acc, idx_v, vals_v, sems):
        cid = lax.axis_index("c")
        sid = lax.axis_index("s")
        fi = sid // 8
        f8 = sid % 8

        zeros16 = jnp.zeros((L,), jnp.float32)

        def zbody(t, carry):
            acc[pl.ds(t * L, L)] = zeros16
            return carry
        lax.fori_loop(0, NUM_NODES // L, zbody, 0)

        def issue(idx_hbm, attr_hbm, it, b):
            pltpu.async_copy(
                idx_hbm.at[pl.ds(cid * e_sc + it * K, K)],
                idx_v.at[b], sems.at[b])
            pltpu.async_copy(
                attr_hbm.at[fi, pl.ds(cid * blk_sc + it * KB, KB), f8],
                vals_v.at[b], sems.at[b])

        def drain(idx_hbm, attr_hbm, b):
            pltpu.make_async_copy(
                idx_hbm.at[pl.ds(0, K)], idx_v.at[b], sems.at[b]).wait()
            pltpu.make_async_copy(
                attr_hbm.at[0, pl.ds(0, KB), 0], vals_v.at[b],
                sems.at[b]).wait()

        def run(idx_hbm, attr_hbm):
            for b in range(NBUF):
                issue(idx_hbm, attr_hbm, b, b)

            def body(step, carry):
                for b in range(NBUF):
                    drain(idx_hbm, attr_hbm, b)

                    @plsc.parallel_loop(0, groups, 1, unroll=8)
                    def _(g):
                        gi = idx_v[b, pl.ds(g * L, L)]
                        gv = vals_v[b, g >> 3, pl.ds((g & 7) * L, L)]
                        plsc.addupdate_scatter(acc, [gi], gv)

                    @pl.when(step < steps - 1)
                    def _():
                        issue(idx_hbm, attr_hbm, (step + 1) * NBUF + b, b)
                return carry
            lax.fori_loop(0, steps, body, 0)

        run(idx1_hbm, attr1_hbm)
        run(idx2_hbm, attr2_hbm)

        pltpu.sync_copy(acc, out_hbm.at[cid, sid])

    return k


@functools.lru_cache(maxsize=None)
def _tc_add():
    blk = 6400
    return pl.pallas_call(
        _add_body,
        grid=(pl.cdiv(NUM_NODES, blk),),
        in_specs=[pl.BlockSpec((2, D, blk), lambda i: (0, 0, i))],
        out_specs=pl.BlockSpec((D, blk), lambda i: (0, i)),
        out_shape=jax.ShapeDtypeStruct((D, NUM_NODES), jnp.float32),
    )


def kernel(edge_attr, edge_attr2, edge_index, edge_index2, num_nodes):
    E = edge_attr.shape[0]
    nblk = E // 128

    def as_storage_view(attr):
        return attr.T.reshape(2, 8, nblk, 128).transpose(0, 2, 1, 3)

    idx1 = edge_index[0].astype(jnp.int32)
    idx2 = edge_index2[0].astype(jnp.int32)
    attr1 = as_storage_view(edge_attr)
    attr2 = as_storage_view(edge_attr2)

    partial = _sc_scatter(E)(idx1, attr1, idx2, attr2)
    out_t = _tc_add()(partial)
    return out_t.T


# speedup vs baseline: 1.0801x; 1.0091x over previous
import functools

import jax
import jax.numpy as jnp
from jax import lax
from jax.experimental import pallas as pl
from jax.experimental.pallas import tpu as pltpu
from jax.experimental.pallas import tpu_sc as plsc

NUM_NODES = 100000
D = 16
NC = 2
NS = 16
L = 16
KB = 25
K = KB * 128
NBUF = 4


def _add_body(p_ref, o_ref):
    o_ref[...] = p_ref[0] + p_ref[1]


@functools.lru_cache(maxsize=None)
def _sc_scatter(E):
    nblk = E // 128
    blk_sc = nblk // NC
    iters = blk_sc // KB
    steps = iters // NBUF
    e_sc = E // NC
    groups = K // L

    @functools.partial(
        pl.kernel,
        out_type=jax.ShapeDtypeStruct((NC, NS, NUM_NODES), jnp.float32),
        mesh=plsc.VectorSubcoreMesh(core_axis_name="c", subcore_axis_name="s"),
        compiler_params=pltpu.CompilerParams(
            use_tc_tiling_on_sc=False, needs_layout_passes=False),
        scratch_types=[
            pltpu.VMEM((NUM_NODES,), jnp.float32),
            pltpu.VMEM((NBUF, K), jnp.int32),
            pltpu.VMEM((NBUF, KB, 128), jnp.float32),
            pltpu.SemaphoreType.DMA((NBUF,)),
        ],
    )
    def k(idx1_hbm, attr1_hbm, idx2_hbm, attr2_hbm, out_hbm,
          acc, idx_v, vals_v, sems):
        cid = lax.axis_index("c")
        sid = lax.axis_index("s")
        fi = sid // 8
        f8 = sid % 8

        zeros16 = jnp.zeros((L,), jnp.float32)

        def issue(idx_hbm, attr_hbm, it, b):
            pltpu.async_copy(
                idx_hbm.at[pl.ds(cid * e_sc + it * K, K)],
                idx_v.at[b], sems.at[b])
            pltpu.async_copy(
                attr_hbm.at[fi, pl.ds(cid * blk_sc + it * KB, KB), f8],
                vals_v.at[b], sems.at[b])

        def drain(idx_hbm, attr_hbm, b):
            pltpu.make_async_copy(
                idx_hbm.at[pl.ds(0, K)], idx_v.at[b], sems.at[b]).wait()
            pltpu.make_async_copy(
                attr_hbm.at[0, pl.ds(0, KB), 0], vals_v.at[b],
                sems.at[b]).wait()

        def run(idx_hbm, attr_hbm, prologue=None):
            for b in range(NBUF):
                issue(idx_hbm, attr_hbm, b, b)
            if prologue is not None:
                prologue()

            def body(step, carry):
                for b in range(NBUF):
                    drain(idx_hbm, attr_hbm, b)

                    @plsc.parallel_loop(0, groups, 1, unroll=8)
                    def _(g):
                        gi = idx_v[b, pl.ds(g * L, L)]
                        gv = vals_v[b, g >> 3, pl.ds((g & 7) * L, L)]
                        plsc.addupdate_scatter(acc, [gi], gv)

                    @pl.when(step < steps - 1)
                    def _():
                        issue(idx_hbm, attr_hbm, (step + 1) * NBUF + b, b)
                return carry
            lax.fori_loop(0, steps, body, 0)

        def zero_acc():
            def zbody(t, carry):
                acc[pl.ds(t * L, L)] = zeros16
                return carry
            lax.fori_loop(0, NUM_NODES // L, zbody, 0)

        run(idx1_hbm, attr1_hbm, prologue=zero_acc)
        run(idx2_hbm, attr2_hbm)

        pltpu.sync_copy(acc, out_hbm.at[cid, sid])

    return k


@functools.lru_cache(maxsize=None)
def _tc_add():
    blk = 6400
    return pl.pallas_call(
        _add_body,
        grid=(pl.cdiv(NUM_NODES, blk),),
        in_specs=[pl.BlockSpec((2, D, blk), lambda i: (0, 0, i))],
        out_specs=pl.BlockSpec((D, blk), lambda i: (0, i)),
        out_shape=jax.ShapeDtypeStruct((D, NUM_NODES), jnp.float32),
    )


def kernel(edge_attr, edge_attr2, edge_index, edge_index2, num_nodes):
    E = edge_attr.shape[0]
    nblk = E // 128

    def as_storage_view(attr):
        return attr.T.reshape(2, 8, nblk, 128).transpose(0, 2, 1, 3)

    idx1 = edge_index[0].astype(jnp.int32)
    idx2 = edge_index2[0].astype(jnp.int32)
    attr1 = as_storage_view(edge_attr)
    attr2 = as_storage_view(edge_attr2)

    partial = _sc_scatter(E)(idx1, attr1, idx2, attr2)
    out_t = _tc_add()(partial)
    return out_t.T


# speedup vs baseline: 1.1310x; 1.0471x over previous
import functools

import jax
import jax.numpy as jnp
from jax import lax
from jax.experimental import pallas as pl
from jax.experimental.pallas import tpu as pltpu
from jax.experimental.pallas import tpu_sc as plsc

NUM_NODES = 100000
D = 16
NC = 2
NS = 16
L = 16
KB = 25
K = KB * 128
NBUF = 4


def _add_body(p_ref, o_ref):
    o_ref[...] = p_ref[0] + p_ref[1]


@functools.lru_cache(maxsize=None)
def _sc_scatter(E):
    nblk = E // 128
    blk_sc = nblk // NC
    iters = blk_sc // KB
    steps = iters // NBUF
    e_sc = E // NC
    groups = K // L

    @functools.partial(
        pl.kernel,
        out_type=jax.ShapeDtypeStruct((NC, NS, NUM_NODES), jnp.float32),
        mesh=plsc.VectorSubcoreMesh(core_axis_name="c", subcore_axis_name="s"),
        compiler_params=pltpu.CompilerParams(
            use_tc_tiling_on_sc=False, needs_layout_passes=False),
        scratch_types=[
            pltpu.VMEM((NUM_NODES,), jnp.float32),
            pltpu.VMEM((NBUF, KB, 128), jnp.int32),
            pltpu.VMEM((NBUF, KB, 128), jnp.float32),
            pltpu.SemaphoreType.DMA((NBUF,)),
        ],
    )
    def k(idx1_hbm, attr1_hbm, idx2_hbm, attr2_hbm, out_hbm,
          acc, idx_v, vals_v, sems):
        cid = lax.axis_index("c")
        sid = lax.axis_index("s")
        fi = sid // 8
        f8 = sid % 8

        zeros16 = jnp.zeros((L,), jnp.float32)

        def issue(idx_hbm, attr_hbm, it, b):
            pltpu.async_copy(
                idx_hbm.at[pl.ds(cid * blk_sc + it * KB, KB), 0],
                idx_v.at[b], sems.at[b])
            pltpu.async_copy(
                attr_hbm.at[fi, pl.ds(cid * blk_sc + it * KB, KB), f8],
                vals_v.at[b], sems.at[b])

        def drain(idx_hbm, attr_hbm, b):
            pltpu.make_async_copy(
                idx_hbm.at[pl.ds(0, KB), 0], idx_v.at[b], sems.at[b]).wait()
            pltpu.make_async_copy(
                attr_hbm.at[0, pl.ds(0, KB), 0], vals_v.at[b],
                sems.at[b]).wait()

        def run(idx_hbm, attr_hbm, prologue=None):
            for b in range(NBUF):
                issue(idx_hbm, attr_hbm, b, b)
            if prologue is not None:
                prologue()

            def body(step, carry):
                for b in range(NBUF):
                    drain(idx_hbm, attr_hbm, b)

                    @plsc.parallel_loop(0, groups, 1, unroll=8)
                    def _(g):
                        gi = idx_v[b, g >> 3, pl.ds((g & 7) * L, L)]
                        gv = vals_v[b, g >> 3, pl.ds((g & 7) * L, L)]
                        plsc.addupdate_scatter(acc, [gi], gv)

                    @pl.when(step < steps - 1)
                    def _():
                        issue(idx_hbm, attr_hbm, (step + 1) * NBUF + b, b)
                return carry
            lax.fori_loop(0, steps, body, 0)

        def zero_acc():
            def zbody(t, carry):
                acc[pl.ds(t * L, L)] = zeros16
                return carry
            lax.fori_loop(0, NUM_NODES // L, zbody, 0)

        run(idx1_hbm, attr1_hbm, prologue=zero_acc)
        run(idx2_hbm, attr2_hbm)

        pltpu.sync_copy(acc, out_hbm.at[cid, sid])

    return k


@functools.lru_cache(maxsize=None)
def _tc_add():
    blk = 6400
    return pl.pallas_call(
        _add_body,
        grid=(pl.cdiv(NUM_NODES, blk),),
        in_specs=[pl.BlockSpec((2, D, blk), lambda i: (0, 0, i))],
        out_specs=pl.BlockSpec((D, blk), lambda i: (0, i)),
        out_shape=jax.ShapeDtypeStruct((D, NUM_NODES), jnp.float32),
    )


def kernel(edge_attr, edge_attr2, edge_index, edge_index2, num_nodes):
    E = edge_attr.shape[0]
    nblk = E // 128

    def as_storage_view(attr):
        return attr.T.reshape(2, 8, nblk, 128).transpose(0, 2, 1, 3)

    def as_index_view(ei):
        return ei.astype(jnp.int32).T.reshape(nblk, 128, 2).transpose(0, 2, 1)

    idx1 = as_index_view(edge_index)
    idx2 = as_index_view(edge_index2)
    attr1 = as_storage_view(edge_attr)
    attr2 = as_storage_view(edge_attr2)

    partial = _sc_scatter(E)(idx1, attr1, idx2, attr2)
    out_t = _tc_add()(partial)
    return out_t.T
